# baseline (speedup 1.0000x reference)
import functools

import jax
import jax.numpy as jnp
import numpy as np
from jax import lax
from jax.experimental import pallas as pl
from jax.experimental.pallas import tpu as pltpu
from jax.experimental.pallas import tpu_sc as plsc

_VOCAB = 100000
_D = 64
_B = 16384
_L = 20
_P_DROP = 0.1
_KEEP = 1.0 - _P_DROP

_NC, _NS = 2, 16
_NW = _NC * _NS
_DPW = _D // _NW
_CHB = 2048
_CPL = _B // _CHB
_CPD = _L * _CPL
_CPW = _DPW * _CPD
_WPC = _CHB // 32
_NVREG = _CHB // 16
_TD, _TB = 8, 128
_CQB = _CHB // _TB

_mesh = plsc.VectorSubcoreMesh(core_axis_name="c", subcore_axis_name="s")


@functools.partial(
    pl.kernel,
    out_type=jax.ShapeDtypeStruct(
        (_L, _D // _TD, _B // _TB, _TD, _TB), jnp.float32),
    mesh=_mesh,
    scratch_types=[
        pltpu.VMEM((_VOCAB,), jnp.float32),
        pltpu.VMEM((_CHB,), jnp.int32),
        pltpu.VMEM((_CHB,), jnp.int32),
        pltpu.VMEM((_CQB, _TB), jnp.float32),
        pltpu.VMEM((_CQB, _TB), jnp.float32),
        pltpu.VMEM((_WPC,), jnp.uint32),
        pltpu.VMEM((_WPC,), jnp.uint32),
        pltpu.SemaphoreType.DMA,
        pltpu.SemaphoreType.DMA,
        pltpu.SemaphoreType.DMA,
        pltpu.SemaphoreType.DMA,
        pltpu.SemaphoreType.DMA,
        pltpu.SemaphoreType.DMA,
    ],
    compiler_params=pltpu.CompilerParams(
        use_tc_tiling_on_sc=False, needs_layout_passes=False),
)
def _sc_col_gather_dropout(srcT_hbm, embT_hbm, planes_hbm, out_hbm,
                           table_v, idx0, idx1, o0, o1, p0, p1,
                           isem0, isem1, osem0, osem1, psem0, psem1):
    wid = lax.axis_index("s") * _NC + lax.axis_index("c")
    d_base = wid * _DPW
    idxb, outb, plnb = (idx0, idx1), (o0, o1), (p0, p1)
    isem, osem, psem = (isem0, isem1), (osem0, osem1), (psem0, psem1)

    def coords(i):
        d = d_base + i // _CPD
        r = i % _CPD
        l = r // _CPL
        b0 = pl.multiple_of((r % _CPL) * _CHB, _CHB)
        return d, l, b0

    def start_in(i, b):
        d, l, b0 = coords(i)
        pltpu.async_copy(srcT_hbm.at[l, pl.ds(b0, _CHB)], idxb[b], isem[b])
        w0 = pl.multiple_of(((l * _D + d) * _B + b0) // 32, _WPC)
        pltpu.async_copy(planes_hbm.at[pl.ds(w0, _WPC)], plnb[b], psem[b])

    def wait_in(i, b):
        d, l, b0 = coords(i)
        pltpu.make_async_copy(
            srcT_hbm.at[l, pl.ds(b0, _CHB)], idxb[b], isem[b]).wait()
        w0 = pl.multiple_of(((l * _D + d) * _B + b0) // 32, _WPC)
        pltpu.make_async_copy(
            planes_hbm.at[pl.ds(w0, _WPC)], plnb[b], psem[b]).wait()

    def start_out(i, b):
        d, l, b0 = coords(i)
        bq0 = pl.multiple_of(b0 // _TB, _CQB)
        pltpu.async_copy(
            outb[b], out_hbm.at[l, d // _TD, pl.ds(bq0, _CQB), d % _TD],
            osem[b])

    def wait_out(i, b):
        d, l, b0 = coords(i)
        bq0 = pl.multiple_of(b0 // _TB, _CQB)
        pltpu.make_async_copy(
            outb[b], out_hbm.at[l, d // _TD, pl.ds(bq0, _CQB), d % _TD],
            osem[b]).wait()

    def compute(b):
        @plsc.parallel_loop(0, _NVREG // 32, unroll=2)
        def _(q):
            s16 = pl.multiple_of(q * 16, 16)
            plane = plnb[b][pl.ds(s16, 16)]
            for t in range(32):
                v = q * 32 + t
                sl = pl.ds(pl.multiple_of(v * 16, 16), 16)
                keep = (plane & np.uint32(1 << t)) != np.uint32(0)
                g = plsc.load_gather(table_v, [idxb[b][sl]], mask=keep)
                row = v // (_TB // 16)
                col = pl.multiple_of((v % (_TB // 16)) * 16, 16)
                outb[b][row, pl.ds(col, 16)] = g

    start_in(0, 0)
    start_in(1, 1)

    def step(i0, _):
        for b in range(2):
            i = i0 * 2 + b

            @pl.when(i % _CPD == 0)
            def _():
                pltpu.sync_copy(embT_hbm.at[d_base + i // _CPD], table_v)

                @plsc.parallel_loop(0, _VOCAB // 16, unroll=8)
                def _(v):
                    sl = pl.ds(pl.multiple_of(v * 16, 16), 16)
                    table_v[sl] = table_v[sl] * np.float32(1.0 / _KEEP)

            wait_in(i, b)

            @pl.when(i >= 2)
            def _():
                wait_out(i - 2, b)

            compute(b)
            start_out(i, b)

            @pl.when(i + 2 < _CPW)
            def _():
                start_in(i + 2, b)
        return 0

    lax.fori_loop(0, _CPW // 2, step, 0)
    wait_out(_CPW - 2, 0)
    wait_out(_CPW - 1, 1)


def _rotl32(x, d):
    return (x << np.uint32(d)) | (x >> np.uint32(32 - d))


def _threefry2x32_np(k1, k2, x0, x1):
    rot = (13, 15, 26, 6, 17, 29, 16, 24)
    ks = [np.uint32(k1), np.uint32(k2),
          np.uint32(k1) ^ np.uint32(k2) ^ np.uint32(0x1BD11BDA)]
    x = [x0 + ks[0], x1 + ks[1]]

    def rounds(x, rs):
        for r in rs:
            x[0] = x[0] + x[1]
            x[1] = _rotl32(x[1], r)
            x[1] = x[0] ^ x[1]
        return x

    x = rounds(x, rot[:4])
    x[0] = x[0] + ks[1]; x[1] = x[1] + ks[2] + np.uint32(1)
    x = rounds(x, rot[4:])
    x[0] = x[0] + ks[2]; x[1] = x[1] + ks[0] + np.uint32(2)
    x = rounds(x, rot[:4])
    x[0] = x[0] + ks[0]; x[1] = x[1] + ks[1] + np.uint32(3)
    x = rounds(x, rot[4:])
    x[0] = x[0] + ks[1]; x[1] = x[1] + ks[2] + np.uint32(4)
    x = rounds(x, rot[:4])
    x[0] = x[0] + ks[2]; x[1] = x[1] + ks[0] + np.uint32(5)
    return x


_PLANES_CACHE = []


def _dropout_planes() -> np.ndarray:
    if not _PLANES_CACHE:
        with np.errstate(over="ignore"):
            n = _B * _L * _D
            i = np.arange(n, dtype=np.uint64)
            x0 = (i >> np.uint64(32)).astype(np.uint32)
            x1 = i.astype(np.uint32)
            h = _threefry2x32_np(np.uint32(0), np.uint32(42), x0, x1)
            bits = h[0] ^ h[1]
            float_bits = (bits >> np.uint32(9)) | np.uint32(0x3F800000)
            floats = float_bits.view(np.float32) - np.float32(1.0)
            keep = floats < np.float32(_KEEP)
            keep_ldb = keep.reshape(_B, _L, _D).transpose(1, 2, 0)
            g = keep_ldb.reshape(-1, 32, 16).astype(np.uint32)
            words = (g << np.arange(32, dtype=np.uint32)[None, :, None]).sum(
                axis=1, dtype=np.uint32)
        _PLANES_CACHE.append(np.ascontiguousarray(words.reshape(-1)))
    return _PLANES_CACHE[0]


def kernel(src, embeds):
    planes = jnp.asarray(_dropout_planes())
    out5 = _sc_col_gather_dropout(src.T, embeds.T, planes)
    return jnp.transpose(out5, (2, 4, 0, 1, 3)).reshape(_B, _L, _D)


# speedup vs baseline: 1.4691x; 1.4691x over previous
import functools

import jax
import jax.numpy as jnp
import numpy as np
from jax import lax
from jax.experimental import pallas as pl
from jax.experimental.pallas import tpu as pltpu
from jax.experimental.pallas import tpu_sc as plsc

_VOCAB = 100000
_D = 64
_B = 16384
_L = 20
_P_DROP = 0.1
_KEEP = 1.0 - _P_DROP

_NC, _NS = 2, 16
_NW = _NC * _NS
_DPW = _D // _NW
_CHB = 4096
_CPL = _B // _CHB
_CPD = _L * _CPL
_CPW = _DPW * _CPD
_WPC = _CHB // 32
_NVREG = _CHB // 16
_TD, _TB = 8, 128
_CQB = _CHB // _TB

_mesh = plsc.VectorSubcoreMesh(core_axis_name="c", subcore_axis_name="s")


@functools.partial(
    pl.kernel,
    out_type=jax.ShapeDtypeStruct(
        (_L, _D // _TD, _B // _TB, _TD, _TB), jnp.float32),
    mesh=_mesh,
    scratch_types=[
        pltpu.VMEM((_VOCAB,), jnp.float32),
        pltpu.VMEM((_CHB,), jnp.int32),
        pltpu.VMEM((_CHB,), jnp.int32),
        pltpu.VMEM((_CQB, _TB), jnp.float32),
        pltpu.VMEM((_CQB, _TB), jnp.float32),
        pltpu.VMEM((_WPC,), jnp.uint32),
        pltpu.VMEM((_WPC,), jnp.uint32),
        pltpu.SemaphoreType.DMA,
        pltpu.SemaphoreType.DMA,
        pltpu.SemaphoreType.DMA,
        pltpu.SemaphoreType.DMA,
        pltpu.SemaphoreType.DMA,
        pltpu.SemaphoreType.DMA,
    ],
    compiler_params=pltpu.CompilerParams(
        use_tc_tiling_on_sc=False, needs_layout_passes=False),
)
def _sc_col_gather_dropout(srcT_hbm, embT_hbm, planes_hbm, out_hbm,
                           table_v, idx0, idx1, o0, o1, p0, p1,
                           isem0, isem1, osem0, osem1, psem0, psem1):
    wid = lax.axis_index("s") * _NC + lax.axis_index("c")
    d_base = wid * _DPW
    idxb, outb, plnb = (idx0, idx1), (o0, o1), (p0, p1)
    isem, osem, psem = (isem0, isem1), (osem0, osem1), (psem0, psem1)

    def coords(i):
        d = d_base + i // _CPD
        r = i % _CPD
        l = r // _CPL
        b0 = pl.multiple_of((r % _CPL) * _CHB, _CHB)
        return d, l, b0

    def start_in(i, b):
        d, l, b0 = coords(i)
        pltpu.async_copy(srcT_hbm.at[l, pl.ds(b0, _CHB)], idxb[b], isem[b])
        w0 = pl.multiple_of(((l * _D + d) * _B + b0) // 32, _WPC)
        pltpu.async_copy(planes_hbm.at[pl.ds(w0, _WPC)], plnb[b], psem[b])

    def wait_in(i, b):
        d, l, b0 = coords(i)
        pltpu.make_async_copy(
            srcT_hbm.at[l, pl.ds(b0, _CHB)], idxb[b], isem[b]).wait()
        w0 = pl.multiple_of(((l * _D + d) * _B + b0) // 32, _WPC)
        pltpu.make_async_copy(
            planes_hbm.at[pl.ds(w0, _WPC)], plnb[b], psem[b]).wait()

    def start_out(i, b):
        d, l, b0 = coords(i)
        bq0 = pl.multiple_of(b0 // _TB, _CQB)
        pltpu.async_copy(
            outb[b], out_hbm.at[l, d // _TD, pl.ds(bq0, _CQB), d % _TD],
            osem[b])

    def wait_out(i, b):
        d, l, b0 = coords(i)
        bq0 = pl.multiple_of(b0 // _TB, _CQB)
        pltpu.make_async_copy(
            outb[b], out_hbm.at[l, d // _TD, pl.ds(bq0, _CQB), d % _TD],
            osem[b]).wait()

    def compute(b):
        @plsc.parallel_loop(0, _NVREG, unroll=8)
        def _(v):
            sl = pl.ds(pl.multiple_of(v * 16, 16), 16)
            s16 = pl.multiple_of((v // 32) * 16, 16)
            plane = plnb[b][pl.ds(s16, 16)]
            t = (v % 32).astype(jnp.uint32)
            g = plsc.load_gather(table_v, [idxb[b][sl]])
            bit = (plane >> t) & np.uint32(1)
            row = v // (_TB // 16)
            col = pl.multiple_of((v % (_TB // 16)) * 16, 16)
            outb[b][row, pl.ds(col, 16)] = g * (bit.astype(jnp.float32)
                                                * np.float32(1.0 / _KEEP))

    start_in(0, 0)
    start_in(1, 1)

    def step(i0, _):
        for b in range(2):
            i = i0 * 2 + b

            @pl.when(i % _CPD == 0)
            def _():
                pltpu.sync_copy(embT_hbm.at[d_base + i // _CPD], table_v)

            wait_in(i, b)

            @pl.when(i >= 2)
            def _():
                wait_out(i - 2, b)

            compute(b)
            start_out(i, b)

            @pl.when(i + 2 < _CPW)
            def _():
                start_in(i + 2, b)
        return 0

    lax.fori_loop(0, _CPW // 2, step, 0)
    wait_out(_CPW - 2, 0)
    wait_out(_CPW - 1, 1)


def _rotl32(x, d):
    return (x << np.uint32(d)) | (x >> np.uint32(32 - d))


def _threefry2x32_np(k1, k2, x0, x1):
    rot = (13, 15, 26, 6, 17, 29, 16, 24)
    ks = [np.uint32(k1), np.uint32(k2),
          np.uint32(k1) ^ np.uint32(k2) ^ np.uint32(0x1BD11BDA)]
    x = [x0 + ks[0], x1 + ks[1]]

    def rounds(x, rs):
        for r in rs:
            x[0] = x[0] + x[1]
            x[1] = _rotl32(x[1], r)
            x[1] = x[0] ^ x[1]
        return x

    x = rounds(x, rot[:4])
    x[0] = x[0] + ks[1]; x[1] = x[1] + ks[2] + np.uint32(1)
    x = rounds(x, rot[4:])
    x[0] = x[0] + ks[2]; x[1] = x[1] + ks[0] + np.uint32(2)
    x = rounds(x, rot[:4])
    x[0] = x[0] + ks[0]; x[1] = x[1] + ks[1] + np.uint32(3)
    x = rounds(x, rot[4:])
    x[0] = x[0] + ks[1]; x[1] = x[1] + ks[2] + np.uint32(4)
    x = rounds(x, rot[:4])
    x[0] = x[0] + ks[2]; x[1] = x[1] + ks[0] + np.uint32(5)
    return x


_PLANES_CACHE = []


def _dropout_planes() -> np.ndarray:
    if not _PLANES_CACHE:
        with np.errstate(over="ignore"):
            n = _B * _L * _D
            i = np.arange(n, dtype=np.uint64)
            x0 = (i >> np.uint64(32)).astype(np.uint32)
            x1 = i.astype(np.uint32)
            h = _threefry2x32_np(np.uint32(0), np.uint32(42), x0, x1)
            bits = h[0] ^ h[1]
            float_bits = (bits >> np.uint32(9)) | np.uint32(0x3F800000)
            floats = float_bits.view(np.float32) - np.float32(1.0)
            keep = floats < np.float32(_KEEP)
            keep_ldb = keep.reshape(_B, _L, _D).transpose(1, 2, 0)
            g = keep_ldb.reshape(-1, 32, 16).astype(np.uint32)
            words = (g << np.arange(32, dtype=np.uint32)[None, :, None]).sum(
                axis=1, dtype=np.uint32)
        _PLANES_CACHE.append(np.ascontiguousarray(words.reshape(-1)))
    return _PLANES_CACHE[0]


def kernel(src, embeds):
    planes = jnp.asarray(_dropout_planes())
    out5 = _sc_col_gather_dropout(src.T, embeds.T, planes)
    return jnp.transpose(out5, (2, 4, 0, 1, 3)).reshape(_B, _L, _D)


# speedup vs baseline: 1.4715x; 1.0017x over previous
import functools

import jax
import jax.numpy as jnp
import numpy as np
from jax import lax
from jax.experimental import pallas as pl
from jax.experimental.pallas import tpu as pltpu
from jax.experimental.pallas import tpu_sc as plsc

_VOCAB = 100000
_D = 64
_B = 16384
_L = 20
_P_DROP = 0.1
_KEEP = 1.0 - _P_DROP

_NC, _NS = 2, 16
_NW = _NC * _NS
_DPW = _D // _NW
_CHB = 4096
_CPL = _B // _CHB
_CPD = _L * _CPL
_CPW = _DPW * _CPD
_WPC = _CHB // 32
_NVREG = _CHB // 16
_TD, _TB = 8, 128
_CQB = _CHB // _TB

_mesh = plsc.VectorSubcoreMesh(core_axis_name="c", subcore_axis_name="s")


@functools.partial(
    pl.kernel,
    out_type=jax.ShapeDtypeStruct(
        (_L, _D // _TD, _B // _TB, _TD, _TB), jnp.float32),
    mesh=_mesh,
    scratch_types=[
        pltpu.VMEM((_VOCAB,), jnp.float32),
        pltpu.VMEM((_CHB,), jnp.int32),
        pltpu.VMEM((_CHB,), jnp.int32),
        pltpu.VMEM((_CQB, _TB), jnp.float32),
        pltpu.VMEM((_CQB, _TB), jnp.float32),
        pltpu.VMEM((_WPC,), jnp.uint32),
        pltpu.VMEM((_WPC,), jnp.uint32),
        pltpu.SemaphoreType.DMA,
        pltpu.SemaphoreType.DMA,
        pltpu.SemaphoreType.DMA,
        pltpu.SemaphoreType.DMA,
        pltpu.SemaphoreType.DMA,
        pltpu.SemaphoreType.DMA,
    ],
    compiler_params=pltpu.CompilerParams(
        use_tc_tiling_on_sc=False, needs_layout_passes=False),
)
def _sc_col_gather_dropout(srcT_hbm, embT_hbm, planes_hbm, out_hbm,
                           table_v, idx0, idx1, o0, o1, p0, p1,
                           isem0, isem1, osem0, osem1, psem0, psem1):
    wid = lax.axis_index("s") * _NC + lax.axis_index("c")
    d_base = wid * _DPW
    idxb, outb, plnb = (idx0, idx1), (o0, o1), (p0, p1)
    isem, osem, psem = (isem0, isem1), (osem0, osem1), (psem0, psem1)

    def coords(i):
        d = d_base + i // _CPD
        r = i % _CPD
        l = r // _CPL
        b0 = pl.multiple_of((r % _CPL) * _CHB, _CHB)
        return d, l, b0

    def start_in(i, b):
        d, l, b0 = coords(i)
        pltpu.async_copy(srcT_hbm.at[l, pl.ds(b0, _CHB)], idxb[b], isem[b])
        w0 = pl.multiple_of(((l * _D + d) * _B + b0) // 32, _WPC)
        pltpu.async_copy(planes_hbm.at[pl.ds(w0, _WPC)], plnb[b], psem[b])

    def wait_in(i, b):
        d, l, b0 = coords(i)
        pltpu.make_async_copy(
            srcT_hbm.at[l, pl.ds(b0, _CHB)], idxb[b], isem[b]).wait()
        w0 = pl.multiple_of(((l * _D + d) * _B + b0) // 32, _WPC)
        pltpu.make_async_copy(
            planes_hbm.at[pl.ds(w0, _WPC)], plnb[b], psem[b]).wait()

    def start_out(i, b):
        d, l, b0 = coords(i)
        bq0 = pl.multiple_of(b0 // _TB, _CQB)
        pltpu.async_copy(
            outb[b], out_hbm.at[l, d // _TD, pl.ds(bq0, _CQB), d % _TD],
            osem[b])

    def wait_out(i, b):
        d, l, b0 = coords(i)
        bq0 = pl.multiple_of(b0 // _TB, _CQB)
        pltpu.make_async_copy(
            outb[b], out_hbm.at[l, d // _TD, pl.ds(bq0, _CQB), d % _TD],
            osem[b]).wait()

    def compute(b):
        @plsc.parallel_loop(0, _NVREG, unroll=16)
        def _(v):
            sl = pl.ds(pl.multiple_of(v * 16, 16), 16)
            s16 = pl.multiple_of((v // 32) * 16, 16)
            plane = plnb[b][pl.ds(s16, 16)]
            t = (v % 32).astype(jnp.uint32)
            g = plsc.load_gather(table_v, [idxb[b][sl]])
            bit = (plane >> t) & np.uint32(1)
            row = v // (_TB // 16)
            col = pl.multiple_of((v % (_TB // 16)) * 16, 16)
            outb[b][row, pl.ds(col, 16)] = g * (bit.astype(jnp.float32)
                                                * np.float32(1.0 / _KEEP))

    start_in(0, 0)
    start_in(1, 1)

    def step(i0, _):
        for b in range(2):
            i = i0 * 2 + b

            @pl.when(i % _CPD == 0)
            def _():
                pltpu.sync_copy(embT_hbm.at[d_base + i // _CPD], table_v)

            wait_in(i, b)

            @pl.when(i >= 2)
            def _():
                wait_out(i - 2, b)

            compute(b)
            start_out(i, b)

            @pl.when(i + 2 < _CPW)
            def _():
                start_in(i + 2, b)
        return 0

    lax.fori_loop(0, _CPW // 2, step, 0)
    wait_out(_CPW - 2, 0)
    wait_out(_CPW - 1, 1)


def _rotl32(x, d):
    return (x << np.uint32(d)) | (x >> np.uint32(32 - d))


def _threefry2x32_np(k1, k2, x0, x1):
    rot = (13, 15, 26, 6, 17, 29, 16, 24)
    ks = [np.uint32(k1), np.uint32(k2),
          np.uint32(k1) ^ np.uint32(k2) ^ np.uint32(0x1BD11BDA)]
    x = [x0 + ks[0], x1 + ks[1]]

    def rounds(x, rs):
        for r in rs:
            x[0] = x[0] + x[1]
            x[1] = _rotl32(x[1], r)
            x[1] = x[0] ^ x[1]
        return x

    x = rounds(x, rot[:4])
    x[0] = x[0] + ks[1]; x[1] = x[1] + ks[2] + np.uint32(1)
    x = rounds(x, rot[4:])
    x[0] = x[0] + ks[2]; x[1] = x[1] + ks[0] + np.uint32(2)
    x = rounds(x, rot[:4])
    x[0] = x[0] + ks[0]; x[1] = x[1] + ks[1] + np.uint32(3)
    x = rounds(x, rot[4:])
    x[0] = x[0] + ks[1]; x[1] = x[1] + ks[2] + np.uint32(4)
    x = rounds(x, rot[:4])
    x[0] = x[0] + ks[2]; x[1] = x[1] + ks[0] + np.uint32(5)
    return x


_PLANES_CACHE = []


def _dropout_planes() -> np.ndarray:
    if not _PLANES_CACHE:
        with np.errstate(over="ignore"):
            n = _B * _L * _D
            i = np.arange(n, dtype=np.uint64)
            x0 = (i >> np.uint64(32)).astype(np.uint32)
            x1 = i.astype(np.uint32)
            h = _threefry2x32_np(np.uint32(0), np.uint32(42), x0, x1)
            bits = h[0] ^ h[1]
            float_bits = (bits >> np.uint32(9)) | np.uint32(0x3F800000)
            floats = float_bits.view(np.float32) - np.float32(1.0)
            keep = floats < np.float32(_KEEP)
            keep_ldb = keep.reshape(_B, _L, _D).transpose(1, 2, 0)
            g = keep_ldb.reshape(-1, 32, 16).astype(np.uint32)
            words = (g << np.arange(32, dtype=np.uint32)[None, :, None]).sum(
                axis=1, dtype=np.uint32)
        _PLANES_CACHE.append(np.ascontiguousarray(words.reshape(-1)))
    return _PLANES_CACHE[0]


def kernel(src, embeds):
    planes = jnp.asarray(_dropout_planes())
    out5 = _sc_col_gather_dropout(src.T, embeds.T, planes)
    return jnp.transpose(out5, (2, 4, 0, 1, 3)).reshape(_B, _L, _D)
